# Initial kernel scaffold; baseline (speedup 1.0000x reference)
import jax
import jax.numpy as jnp
from jax.experimental import pallas as pl


def kernel(x, edge_index, W1, b1, W2, b2):
    raise NotImplementedError("write your pallas kernel here")


# speedup vs baseline: 14.1029x; 14.1029x over previous
import functools

import jax
import jax.numpy as jnp
from jax import lax
from jax.experimental import pallas as pl
from jax.experimental.pallas import tpu as pltpu, tpu_sc as plsc

N_CORES = 2
N_SUBCORES = 16
NW = N_CORES * N_SUBCORES
CHUNK = 80


def _sc_mesh():
    return plsc.VectorSubcoreMesh(
        core_axis_name="c", subcore_axis_name="s",
        num_cores=N_CORES, num_subcores=N_SUBCORES)


_SC_PARAMS = pltpu.CompilerParams(use_tc_tiling_on_sc=False)


def _pad_rows(n):
    per = -(-n // N_SUBCORES)
    per = -(-per // 8) * 8
    return per * N_SUBCORES


def _make_degree_kernel(n_nodes, n_edges):
    ew = n_edges // NW
    nchunks = ew // CHUNK
    npad = _pad_rows(n_nodes)
    nt = npad // N_SUBCORES
    width = 16

    @functools.partial(
        pl.kernel,
        out_type=jax.ShapeDtypeStruct((N_CORES * npad, width), jnp.float32),
        mesh=_sc_mesh(),
        compiler_params=_SC_PARAMS,
        scratch_types=[
            pltpu.VMEM((CHUNK,), jnp.int32),
            pltpu.VMEM((CHUNK, width), jnp.float32),
            pltpu.VMEM_SHARED((npad, width), jnp.float32),
        ],
    )
    def deg_kernel(dst_hbm, ones_hbm, zeros_hbm, out_hbm, idx_v, ones_v, acc_sh):
        c = lax.axis_index("c")
        s = lax.axis_index("s")
        pltpu.sync_copy(zeros_hbm, acc_sh.at[pl.ds(s * nt, nt)])
        pltpu.sync_copy(ones_hbm, ones_v)
        plsc.subcore_barrier()
        base = c * (n_edges // N_CORES) + s * ew

        def chunk(j, carry):
            pltpu.sync_copy(dst_hbm.at[pl.ds(base + j * CHUNK, CHUNK)], idx_v)
            pltpu.sync_copy(ones_v, acc_sh.at[idx_v], add=True)
            return carry

        lax.fori_loop(0, nchunks, chunk, 0)
        plsc.subcore_barrier()
        pltpu.sync_copy(acc_sh.at[pl.ds(s * nt, nt)],
                        out_hbm.at[pl.ds(c * npad + s * nt, nt)])

    return deg_kernel


def _make_scatter_kernel(n_nodes, n_edges, d):
    ew = n_edges // NW
    nchunks = ew // CHUNK
    npad = _pad_rows(n_nodes)
    nt = npad // N_SUBCORES

    @functools.partial(
        pl.kernel,
        out_type=jax.ShapeDtypeStruct((N_CORES * npad, d), jnp.float32),
        mesh=_sc_mesh(),
        compiler_params=_SC_PARAMS,
        scratch_types=[
            pltpu.VMEM((CHUNK,), jnp.int32),
            pltpu.VMEM((CHUNK,), jnp.int32),
            pltpu.VMEM((CHUNK, d), jnp.float32),
            pltpu.VMEM_SHARED((npad, d), jnp.float32),
            pltpu.SemaphoreType.DMA,
        ],
    )
    def scatter_kernel(g_hbm, src_hbm, dst_hbm, zeros_hbm, out_hbm,
                       idxs_v, idxd_v, rows_v, acc_sh, sem):
        c = lax.axis_index("c")
        s = lax.axis_index("s")
        pltpu.sync_copy(zeros_hbm, acc_sh.at[pl.ds(s * nt, nt)])
        plsc.subcore_barrier()
        base = c * (n_edges // N_CORES) + s * ew

        def chunk(j, carry):
            e0 = base + j * CHUNK
            pltpu.sync_copy(src_hbm.at[pl.ds(e0, CHUNK)], idxs_v)
            pltpu.sync_copy(dst_hbm.at[pl.ds(e0, CHUNK)], idxd_v)
            pltpu.async_copy(g_hbm.at[idxs_v], rows_v, sem).wait()
            pltpu.sync_copy(rows_v, acc_sh.at[idxd_v], add=True)
            return carry

        lax.fori_loop(0, nchunks, chunk, 0)
        plsc.subcore_barrier()
        pltpu.sync_copy(acc_sh.at[pl.ds(s * nt, nt)],
                        out_hbm.at[pl.ds(c * npad + s * nt, nt)])

    return scatter_kernel


def _tc_matmul(x, w):
    n, k = x.shape
    m = w.shape[1]
    blk = 2000

    def body(x_ref, w_ref, o_ref):
        o_ref[...] = jnp.dot(x_ref[...], w_ref[...],
                             preferred_element_type=jnp.float32)

    return pl.pallas_call(
        body,
        grid=(n // blk,),
        in_specs=[pl.BlockSpec((blk, k), lambda i: (i, 0)),
                  pl.BlockSpec((k, m), lambda i: (0, 0))],
        out_specs=pl.BlockSpec((blk, m), lambda i: (i, 0)),
        out_shape=jax.ShapeDtypeStruct((n, m), jnp.float32),
    )(x, w)


def _tc_prescale(deg_parts, h1):
    n, d = h1.shape
    blk = 2000
    w = deg_parts.shape[2]

    def body(da_ref, db_ref, h_ref, g_ref, dinv_ref):
        deg = da_ref[0][:, :1] + db_ref[0][:, :1] + 1.0
        dinv = lax.rsqrt(deg)
        dinv_ref[...] = dinv
        g_ref[...] = h_ref[...] * dinv

    return pl.pallas_call(
        body,
        grid=(n // blk,),
        in_specs=[pl.BlockSpec((1, blk, w), lambda i: (0, i, 0)),
                  pl.BlockSpec((1, blk, w), lambda i: (1, i, 0)),
                  pl.BlockSpec((blk, d), lambda i: (i, 0))],
        out_specs=[pl.BlockSpec((blk, d), lambda i: (i, 0)),
                   pl.BlockSpec((blk, 1), lambda i: (i, 0))],
        out_shape=[jax.ShapeDtypeStruct((n, d), jnp.float32),
                   jax.ShapeDtypeStruct((n, 1), jnp.float32)],
    )(deg_parts, deg_parts, h1)


def _tc_mid(s1, g1, dinv, b1, w2):
    n, d = g1.shape
    m = w2.shape[1]
    blk = 2000

    def body(sa_ref, sb_ref, g_ref, dinv_ref, b_ref, w_ref, o_ref):
        dinv = dinv_ref[...]
        h = (sa_ref[0] + sb_ref[0] + g_ref[...]) * dinv + b_ref[...]
        h = jnp.maximum(h, 0.0)
        o_ref[...] = jnp.dot(h, w_ref[...],
                             preferred_element_type=jnp.float32) * dinv

    return pl.pallas_call(
        body,
        grid=(n // blk,),
        in_specs=[pl.BlockSpec((1, blk, d), lambda i: (0, i, 0)),
                  pl.BlockSpec((1, blk, d), lambda i: (1, i, 0)),
                  pl.BlockSpec((blk, d), lambda i: (i, 0)),
                  pl.BlockSpec((blk, 1), lambda i: (i, 0)),
                  pl.BlockSpec((1, d), lambda i: (0, 0)),
                  pl.BlockSpec((d, m), lambda i: (0, 0))],
        out_specs=pl.BlockSpec((blk, m), lambda i: (i, 0)),
        out_shape=jax.ShapeDtypeStruct((n, m), jnp.float32),
    )(s1, s1, g1, dinv, b1, w2)


def _tc_final(s2, g2, dinv, b2):
    n, m = g2.shape
    blk = 2000

    def body(sa_ref, sb_ref, g_ref, dinv_ref, b_ref, o_ref):
        o_ref[...] = ((sa_ref[0] + sb_ref[0] + g_ref[...]) * dinv_ref[...]
                      + b_ref[...])

    return pl.pallas_call(
        body,
        grid=(n // blk,),
        in_specs=[pl.BlockSpec((1, blk, m), lambda i: (0, i, 0)),
                  pl.BlockSpec((1, blk, m), lambda i: (1, i, 0)),
                  pl.BlockSpec((blk, m), lambda i: (i, 0)),
                  pl.BlockSpec((blk, 1), lambda i: (i, 0)),
                  pl.BlockSpec((1, m), lambda i: (0, 0))],
        out_specs=pl.BlockSpec((blk, m), lambda i: (i, 0)),
        out_shape=jax.ShapeDtypeStruct((n, m), jnp.float32),
    )(s2, s2, g2, dinv, b2)


def kernel(x, edge_index, W1, b1, W2, b2):
    n, d_in = x.shape
    hidden = W1.shape[1]
    latent = W2.shape[1]
    e = edge_index.shape[1]
    npad = _pad_rows(n)
    nt = npad // N_SUBCORES
    src = edge_index[0]
    dst = edge_index[1]

    ones_rows = jnp.ones((CHUNK, 16), jnp.float32)
    zeros16 = jnp.zeros((nt, 16), jnp.float32)
    zeros_h = jnp.zeros((nt, hidden), jnp.float32)
    zeros_l = jnp.zeros((nt, latent), jnp.float32)

    deg_parts = _make_degree_kernel(n, e)(dst, ones_rows, zeros16)
    deg_parts = deg_parts.reshape(N_CORES, npad, 16)
    h1 = _tc_matmul(x, W1)
    g1, dinv = _tc_prescale(deg_parts, h1)

    s1 = _make_scatter_kernel(n, e, hidden)(g1, src, dst, zeros_h)
    s1 = s1.reshape(N_CORES, npad, hidden)
    g2 = _tc_mid(s1, g1, dinv, b1.reshape(1, hidden), W2)

    s2 = _make_scatter_kernel(n, e, latent)(g2, src, dst, zeros_l)
    s2 = s2.reshape(N_CORES, npad, latent)
    z = _tc_final(s2, g2, dinv, b2.reshape(1, latent))
    return z
